# baseline (speedup 1.0000x reference)
import jax
import jax.numpy as jnp
from jax import lax
from jax.experimental import pallas as pl
from jax.experimental.pallas import tpu as pltpu

_VOCAB = 1000000
_EMBED_DIM = 64
_HIDDEN = 128
_OUTPUT = 2
_SEQ_LEN = 16384

_K = 50
_S = _VOCAB // _K
_R = 128


def _bitonic_sort(v):
    lane = lax.broadcasted_iota(jnp.int32, (_R, _R), 1)
    sub = lax.broadcasted_iota(jnp.int32, (_R, _R), 0)
    flat = sub * _R + lane

    def roll(a, sh, axis):
        sh = sh % _R
        if axis == 1:
            return jnp.concatenate([a[:, sh:], a[:, :sh]], axis=1)
        return jnp.concatenate([a[sh:, :], a[:sh, :]], axis=0)

    for klen_log in range(1, 15):
        klen = 1 << klen_log
        asc = (flat & klen) == 0
        for j_log in range(klen_log - 1, -1, -1):
            j = 1 << j_log
            if j < _R:
                lower = (lane & j) == 0
                vp = jnp.where(lower, roll(v, j, 1), roll(v, -j, 1))
            else:
                jj = j // _R
                lower = (sub & jj) == 0
                vp = jnp.where(lower, roll(v, jj, 0), roll(v, -jj, 0))
            lo = jnp.minimum(v, vp)
            hi = jnp.maximum(v, vp)
            v = jnp.where(lower == asc, lo, hi)
    return v


def _body(starts_ref, x_ref, table_ref, vwt_ref, vb_ref, wwt_ref, wb_ref,
          o_ref, acc_ref, sorted_v, sorted_s, sem):
    k = pl.program_id(0)

    @pl.when(k == 0)
    def _init():
        acc_ref[...] = jnp.zeros_like(acc_ref)
        sorted_v[...] = _bitonic_sort(x_ref[...])
        copy = pltpu.make_async_copy(sorted_v, sorted_s, sem)
        copy.start()
        copy.wait()

    start = starts_ref[k]
    end = starts_ref[k + 1]
    base = k * _S

    def hit(p, acc):
        row = sorted_s[p // _R, p % _R] - base
        return acc + table_ref[pl.ds(row, 1), :]

    acc_ref[...] = lax.fori_loop(start, end, hit, acc_ref[...])

    @pl.when(k == _K - 1)
    def _finish():
        avg = acc_ref[...] * (1.0 / _SEQ_LEN)
        h = jnp.tanh(
            jnp.dot(avg, vwt_ref[...], precision=lax.Precision.HIGHEST)
            + vb_ref[...]
        )
        o = (
            jnp.dot(h, wwt_ref[...], precision=lax.Precision.HIGHEST)
            + wb_ref[...]
        )
        m = jnp.max(o, axis=1, keepdims=True)
        e = o - m
        lse = jnp.log(jnp.sum(jnp.exp(e), axis=1, keepdims=True))
        o_ref[...] = e - lse


def kernel(x, table, V_w, V_b, W_w, W_b):
    xi = x.astype(jnp.int32)
    slab_bounds = jnp.arange(_K + 1, dtype=jnp.int32) * _S
    starts = jnp.sum(xi[None, :] < slab_bounds[:, None], axis=1).astype(jnp.int32)
    out = pl.pallas_call(
        _body,
        grid_spec=pltpu.PrefetchScalarGridSpec(
            num_scalar_prefetch=1,
            grid=(_K,),
            in_specs=[
                pl.BlockSpec((_R, _R), lambda k, st_s: (0, 0)),
                pl.BlockSpec((_S, _EMBED_DIM), lambda k, st_s: (k, 0)),
                pl.BlockSpec((_EMBED_DIM, _HIDDEN), lambda k, st_s: (0, 0)),
                pl.BlockSpec((1, _HIDDEN), lambda k, st_s: (0, 0)),
                pl.BlockSpec((_HIDDEN, _OUTPUT), lambda k, st_s: (0, 0)),
                pl.BlockSpec((1, _OUTPUT), lambda k, st_s: (0, 0)),
            ],
            out_specs=pl.BlockSpec((1, _OUTPUT), lambda k, st_s: (0, 0)),
            scratch_shapes=[
                pltpu.VMEM((1, _EMBED_DIM), jnp.float32),
                pltpu.VMEM((_R, _R), jnp.int32),
                pltpu.SMEM((_R, _R), jnp.int32),
                pltpu.SemaphoreType.DMA,
            ],
        ),
        out_shape=jax.ShapeDtypeStruct((1, _OUTPUT), jnp.float32),
    )(
        starts,
        xi.reshape(_R, _R),
        table,
        V_w.T,
        V_b.reshape(1, _HIDDEN),
        W_w.T,
        W_b.reshape(1, _OUTPUT),
    )
    return out.reshape(_OUTPUT)
